# baseline (speedup 1.0000x reference)
import functools

import jax
import jax.numpy as jnp
from jax import lax
from jax.experimental import pallas as pl
from jax.experimental.pallas import tpu as pltpu
from jax.experimental.pallas import tpu_sc as plsc

VOCAB = 100000
EMBED_DIM = 64
BATCH = 16384

NUM_CORES = 2
NUM_SUBCORES = 16
NUM_WORKERS = NUM_CORES * NUM_SUBCORES
NUM_PASSES = EMBED_DIM // NUM_WORKERS
ICHUNK = 4096
NCHUNKS = BATCH // ICHUNK
NOC = 2
LANES = 16

_mesh = plsc.VectorSubcoreMesh(core_axis_name="c", subcore_axis_name="s")


@functools.partial(
    pl.kernel,
    mesh=_mesh,
    out_type=jax.ShapeDtypeStruct((EMBED_DIM, BATCH), jnp.float32),
    scratch_types=[
        pltpu.VMEM((VOCAB,), jnp.float32),
        pltpu.VMEM((BATCH,), jnp.int32),
        pltpu.VMEM((NOC, ICHUNK), jnp.float32),
        pltpu.SemaphoreType.DMA,
        pltpu.SemaphoreType.DMA,
        pltpu.SemaphoreType.DMA,
    ],
    compiler_params=pltpu.CompilerParams(
        needs_layout_passes=False,
        skip_device_barrier=True,
        disable_bounds_checks=True,
    ),
)
def _gather_kernel(idx_hbm, tabT_hbm, outT_hbm, row_v, idx_v, oc_v, sem, osem0, osem1):
    wid = lax.axis_index("s") * NUM_CORES + lax.axis_index("c")
    osems = (osem0, osem1)
    idx_cp = pltpu.async_copy(idx_hbm, idx_v, sem)
    row_cp = pltpu.async_copy(tabT_hbm.at[wid, :], row_v, sem)
    idx_cp.wait()
    row_cp.wait()
    out_cps = [None, None]
    for p in range(NUM_PASSES):
        f = p * NUM_WORKERS + wid
        for c in range(NCHUNKS):
            buf = c % NOC
            if out_cps[buf] is not None:
                out_cps[buf].wait()

            @plsc.parallel_loop(0, ICHUNK // LANES, unroll=8)
            def body(j):
                iv = idx_v[pl.ds(c * ICHUNK + j * LANES, LANES)]
                oc_v[buf, pl.ds(j * LANES, LANES)] = plsc.load_gather(row_v, [iv])

            out_cps[buf] = pltpu.async_copy(
                oc_v.at[buf], outT_hbm.at[f, pl.ds(c * ICHUNK, ICHUNK)], osems[buf]
            )
        if p + 1 < NUM_PASSES:
            for b in range(NOC):
                if out_cps[b] is not None:
                    out_cps[b].wait()
                    out_cps[b] = None
            pltpu.sync_copy(tabT_hbm.at[(p + 1) * NUM_WORKERS + wid, :], row_v)
    for b in range(NOC):
        if out_cps[b] is not None:
            out_cps[b].wait()


def kernel(indices, table):
    outT = _gather_kernel(indices.astype(jnp.int32), table.T)
    return outT.T


# speedup vs baseline: 1.0102x; 1.0102x over previous
import functools

import jax
import jax.numpy as jnp
from jax import lax
from jax.experimental import pallas as pl
from jax.experimental.pallas import tpu as pltpu
from jax.experimental.pallas import tpu_sc as plsc

VOCAB = 100000
EMBED_DIM = 64
BATCH = 16384

NUM_CORES = 2
NUM_SUBCORES = 16
NUM_WORKERS = NUM_CORES * NUM_SUBCORES
NUM_PASSES = EMBED_DIM // NUM_WORKERS
ICHUNK = 8192
NCHUNKS = BATCH // ICHUNK
LANES = 16

_mesh = plsc.VectorSubcoreMesh(core_axis_name="c", subcore_axis_name="s")


@functools.partial(
    pl.kernel,
    mesh=_mesh,
    out_type=jax.ShapeDtypeStruct((EMBED_DIM, BATCH), jnp.float32),
    scratch_types=[
        pltpu.VMEM((VOCAB,), jnp.float32),
        pltpu.VMEM((BATCH,), jnp.int32),
        pltpu.VMEM((ICHUNK,), jnp.float32),
        pltpu.SemaphoreType.DMA,
    ],
    compiler_params=pltpu.CompilerParams(needs_layout_passes=False),
)
def _gather_kernel(idx_hbm, tabT_hbm, outT_hbm, row_v, idx_v, oc_v, sem):
    wid = lax.axis_index("s") * NUM_CORES + lax.axis_index("c")
    idx_cp = pltpu.async_copy(idx_hbm, idx_v, sem)
    row_cp = pltpu.async_copy(tabT_hbm.at[wid, :], row_v, sem)
    idx_cp.wait()
    row_cp.wait()
    for p in range(NUM_PASSES):
        f = p * NUM_WORKERS + wid
        for c in range(NCHUNKS):
            @plsc.parallel_loop(0, ICHUNK // LANES, unroll=8)
            def body(j):
                iv = idx_v[pl.ds(c * ICHUNK + j * LANES, LANES)]
                oc_v[pl.ds(j * LANES, LANES)] = plsc.load_gather(row_v, [iv])

            pltpu.sync_copy(oc_v, outT_hbm.at[f, pl.ds(c * ICHUNK, ICHUNK)])
        if p + 1 < NUM_PASSES:
            pltpu.sync_copy(tabT_hbm.at[(p + 1) * NUM_WORKERS + wid, :], row_v)


def kernel(indices, table):
    outT = _gather_kernel(indices.astype(jnp.int32), table.T)
    return outT.T


# speedup vs baseline: 1.0132x; 1.0030x over previous
import functools

import jax
import jax.numpy as jnp
from jax import lax
from jax.experimental import pallas as pl
from jax.experimental.pallas import tpu as pltpu
from jax.experimental.pallas import tpu_sc as plsc

VOCAB = 100000
EMBED_DIM = 64
BATCH = 16384

NUM_CORES = 2
NUM_SUBCORES = 16
NUM_WORKERS = NUM_CORES * NUM_SUBCORES
NUM_PASSES = EMBED_DIM // NUM_WORKERS
ICHUNK = 8192
NCHUNKS = BATCH // ICHUNK
LANES = 16

_mesh = plsc.VectorSubcoreMesh(core_axis_name="c", subcore_axis_name="s")


@functools.partial(
    pl.kernel,
    mesh=_mesh,
    out_type=jax.ShapeDtypeStruct((EMBED_DIM, BATCH), jnp.float32),
    scratch_types=[
        pltpu.VMEM((VOCAB,), jnp.float32),
        pltpu.VMEM((BATCH,), jnp.int32),
        pltpu.VMEM((ICHUNK,), jnp.float32),
        pltpu.SemaphoreType.DMA,
        pltpu.SemaphoreType.DMA,
    ],
    compiler_params=pltpu.CompilerParams(needs_layout_passes=False),
)
def _gather_kernel(idx_hbm, tabT_hbm, outT_hbm, row_v, idx_v, oc_v, sem, osem):
    wid = lax.axis_index("s") * NUM_CORES + lax.axis_index("c")
    idx_cp = pltpu.async_copy(idx_hbm, idx_v, sem)
    row_cp = pltpu.async_copy(tabT_hbm.at[wid, :], row_v, sem)
    idx_cp.wait()
    row_cp.wait()
    for p in range(NUM_PASSES):
        f = p * NUM_WORKERS + wid
        out_cp = None
        for c in range(NCHUNKS):
            if out_cp is not None:
                out_cp.wait()

            @plsc.parallel_loop(0, ICHUNK // LANES, unroll=16)
            def body(j):
                iv = idx_v[pl.ds(c * ICHUNK + j * LANES, LANES)]
                oc_v[pl.ds(j * LANES, LANES)] = plsc.load_gather(row_v, [iv])

            out_cp = pltpu.async_copy(
                oc_v, outT_hbm.at[f, pl.ds(c * ICHUNK, ICHUNK)], osem
            )
        if p + 1 < NUM_PASSES:
            row_cp = pltpu.async_copy(
                tabT_hbm.at[(p + 1) * NUM_WORKERS + wid, :], row_v, sem
            )
            out_cp.wait()
            row_cp.wait()
        else:
            out_cp.wait()


def kernel(indices, table):
    outT = _gather_kernel(indices.astype(jnp.int32), table.T)
    return outT.T
